# baseline (speedup 1.0000x reference)
import jax
import jax.numpy as jnp
from jax import lax
from jax.experimental import pallas as pl
from jax.experimental.pallas import tpu as pltpu
from jax.experimental.pallas import tpu_sc as plsc

NC = 2
NS = 16
NW = NC * NS

B, S, H = 4096, 200, 64
BPW = B // NW
NG = BPW // 16
NBUF = 3
NGRP = S // NBUF


def _body(hid_hbm, iid_hbm, sid_hbm, itab_hbm, stab_hbm, out_hbm,
          stab_v, sid_v, iid_v, inst_v, h0, h1, h2,
          si0, si1, si2, so0, so1, so2, gsem):
    hbufs = (h0, h1, h2)
    sem_in = (si0, si1, si2)
    sem_out = (so0, so1, so2)

    cid = lax.axis_index("c")
    sid = lax.axis_index("s")
    wid = sid * NC + cid
    base = wid * BPW

    pltpu.sync_copy(stab_hbm, stab_v)
    pltpu.sync_copy(sid_hbm.at[:, pl.ds(base, BPW)], sid_v)
    pltpu.sync_copy(iid_hbm.at[pl.ds(base, BPW)], iid_v)
    cps = [pltpu.make_async_copy(itab_hbm.at[h].at[iid_v], inst_v.at[h], gsem)
           for h in range(H)]
    for c in cps:
        c.start()
    for c in cps:
        c.wait()

    def in_copy(s, k):
        return pltpu.make_async_copy(hid_hbm.at[s, :, pl.ds(base, BPW)],
                                     hbufs[k], sem_in[k])

    def out_copy(s, k):
        return pltpu.make_async_copy(hbufs[k],
                                     out_hbm.at[s, :, pl.ds(base, BPW)],
                                     sem_out[k])

    def compute(s, k):
        hb = hbufs[k]
        ids = [sid_v[s, pl.ds(16 * g, 16)] for g in range(NG)]

        def hloop(h, c):
            hsplat = jnp.full((16,), 0, jnp.int32) + h
            for g in range(NG):
                val = plsc.load_gather(stab_v, [ids[g], hsplat])
                plsc.addupdate(hb.at[h, pl.ds(16 * g, 16)],
                               val + inst_v[h, pl.ds(16 * g, 16)])
            return c

        lax.fori_loop(0, H, hloop, 0, unroll=2)

    in_copy(0, 0).start()
    in_copy(1, 1).start()

    in_copy(0, 0).wait()
    compute(0, 0)
    out_copy(0, 0).start()
    in_copy(2, 2).start()

    in_copy(1, 1).wait()
    compute(1, 1)
    out_copy(1, 1).start()
    out_copy(0, 0).wait()
    in_copy(3, 0).start()

    in_copy(2, 2).wait()
    compute(2, 2)
    out_copy(2, 2).start()
    out_copy(1, 1).wait()
    in_copy(4, 1).start()

    def group(g, carry):
        for b in range(NBUF):
            s = NBUF * g + b
            k = b
            k2 = (b + 2) % NBUF
            in_copy(s, k).wait()
            compute(s, k)
            out_copy(s, k).start()
            out_copy(s - 1, k2).wait()
            in_copy(s + 2, k2).start()
        return carry

    lax.fori_loop(1, NGRP, group, 0)

    s = NBUF * NGRP
    in_copy(s, 0).wait()
    compute(s, 0)
    out_copy(s, 0).start()

    in_copy(s + 1, 1).wait()
    compute(s + 1, 1)
    out_copy(s + 1, 1).start()

    out_copy(s - 1, 2).wait()
    out_copy(s, 0).wait()
    out_copy(s + 1, 1).wait()


def kernel(hidden_states, instrument_ids, session_ids, instrument_table,
           session_table):
    hid_t = jnp.transpose(hidden_states, (1, 2, 0))
    sid_t = jnp.transpose(session_ids.astype(jnp.int32), (1, 0))
    itab_t = jnp.transpose(instrument_table, (1, 0))

    k = pl.kernel(
        _body,
        out_type=jax.ShapeDtypeStruct((S, H, B), jnp.float32),
        mesh=plsc.VectorSubcoreMesh(core_axis_name="c", subcore_axis_name="s",
                                    num_cores=NC, num_subcores=NS),
        compiler_params=pltpu.CompilerParams(use_tc_tiling_on_sc=False,
                                             needs_layout_passes=False),
        scratch_types=(
            [pltpu.VMEM((1000, H), jnp.float32),
             pltpu.VMEM((S, BPW), jnp.int32),
             pltpu.VMEM((BPW,), jnp.int32),
             pltpu.VMEM((H, BPW), jnp.float32)]
            + [pltpu.VMEM((H, BPW), jnp.float32) for _ in range(NBUF)]
            + [pltpu.SemaphoreType.DMA for _ in range(2 * NBUF + 1)]
        ),
    )
    out_t = k(hid_t, instrument_ids.astype(jnp.int32), sid_t, itab_t,
              session_table)
    return jnp.transpose(out_t, (2, 0, 1))
